# Initial kernel scaffold; baseline (speedup 1.0000x reference)
import functools

import jax
import jax.numpy as jnp
from jax import lax
from jax.experimental import pallas as pl
from jax.experimental.pallas import tpu as pltpu
from jax.experimental.pallas import tpu_sc as plsc

NC = 2
NS = 16
NW = NC * NS
LANES = 16

F32 = jnp.float32


def _make_sc_hist(n, e):
    e_per = e // NW
    assert e % NW == 0 and e_per % LANES == 0 and n % LANES == 0
    mesh = plsc.VectorSubcoreMesh(core_axis_name="c", subcore_axis_name="s")

    @functools.partial(
        pl.kernel,
        mesh=mesh,
        out_type=jax.ShapeDtypeStruct((NW, n), F32),
        scratch_types=[
            pltpu.VMEM((e_per,), jnp.int32),
            pltpu.VMEM((n,), F32),
        ],
    )
    def hist_kernel(dst_hbm, out_hbm, dst_v, hist_v):
        cid = lax.axis_index("c")
        sid = lax.axis_index("s")
        wid = sid * NC + cid
        pltpu.sync_copy(dst_hbm.at[pl.ds(wid * e_per, e_per)], dst_v)

        zeros = jnp.zeros((LANES,), F32)

        def zero_body(i, carry):
            hist_v[pl.ds(i * LANES, LANES)] = zeros
            return carry

        lax.fori_loop(0, n // LANES, zero_body, 0)

        ones = jnp.ones((LANES,), F32)

        def body(i, carry):
            idx = dst_v[pl.ds(i * LANES, LANES)]
            plsc.addupdate_scatter(hist_v, [idx], ones)
            return carry

        lax.fori_loop(0, e_per // LANES, body, 0)
        pltpu.sync_copy(hist_v, out_hbm.at[wid])

    return hist_kernel


def _tc_prep_body(hist_ref, ins_ref, w_ref, y_ref, dinv_ref):
    deg = jnp.sum(hist_ref[...], axis=0) + 1.0
    dinv = lax.rsqrt(deg)
    x = jnp.dot(ins_ref[...], w_ref[...], preferred_element_type=F32)
    y_ref[...] = x * dinv[:, None]
    dinv_ref[...] = dinv[:, None]


def _tc_prep(hist, ins, W, block_rows):
    n, d_in = ins.shape
    d_out = W.shape[1]
    grid = n // block_rows
    return pl.pallas_call(
        _tc_prep_body,
        grid=(grid,),
        in_specs=[
            pl.BlockSpec((NW, block_rows), lambda i: (0, i)),
            pl.BlockSpec((block_rows, d_in), lambda i: (i, 0)),
            pl.BlockSpec((d_in, d_out), lambda i: (0, 0)),
        ],
        out_specs=[
            pl.BlockSpec((block_rows, d_out), lambda i: (i, 0)),
            pl.BlockSpec((block_rows, 1), lambda i: (i, 0)),
        ],
        out_shape=[
            jax.ShapeDtypeStruct((n, d_out), F32),
            jax.ShapeDtypeStruct((n, 1), F32),
        ],
    )(hist, ins, W)


def _make_sc_agg(n, e, d, chunk, nchunk):
    rows_per_tile = n // NS
    assert nchunk * chunk * NW == e
    mesh = plsc.VectorSubcoreMesh(core_axis_name="c", subcore_axis_name="s")

    @functools.partial(
        pl.kernel,
        mesh=mesh,
        out_type=jax.ShapeDtypeStruct((NC, NS, rows_per_tile, d), F32),
        scratch_types=[
            pltpu.VMEM((nchunk, chunk), jnp.int32),
            pltpu.VMEM((nchunk, chunk), jnp.int32),
            pltpu.VMEM((chunk, d), F32),
            pltpu.VMEM_SHARED((n, d), F32),
            pltpu.SemaphoreType.DMA,
        ],
    )
    def agg_kernel(src_hbm, dst_hbm, y_hbm, zeros_hbm, out_hbm,
                   src_v, dst_v, rows_v, acc_sh, sem):
        cid = lax.axis_index("c")
        sid = lax.axis_index("s")
        wid = sid * NC + cid

        pltpu.sync_copy(src_hbm.at[wid], src_v)
        pltpu.sync_copy(dst_hbm.at[wid], dst_v)

        pltpu.sync_copy(zeros_hbm, acc_sh.at[pl.ds(sid * rows_per_tile,
                                                   rows_per_tile)])
        plsc.subcore_barrier()

        def body(j, carry):
            pltpu.async_copy(y_hbm.at[src_v.at[j]], rows_v, sem).wait()
            pltpu.sync_copy(rows_v, acc_sh.at[dst_v.at[j]], add=True)
            return carry

        lax.fori_loop(0, nchunk, body, 0)

        plsc.subcore_barrier()
        pltpu.sync_copy(
            acc_sh.at[pl.ds(sid * rows_per_tile, rows_per_tile)],
            out_hbm.at[cid, sid],
        )

    return agg_kernel


def _tc_final_body(acc_ref, y_ref, dinv_ref, b_ref, out_ref):
    s = acc_ref[0] + acc_ref[1] + y_ref[...]
    out_ref[...] = jnp.maximum(s * dinv_ref[...] + b_ref[...], 0.0)


def _tc_final(acc, y, dinv, b2, block_rows):
    n, d = y.shape
    grid = n // block_rows
    return pl.pallas_call(
        _tc_final_body,
        grid=(grid,),
        in_specs=[
            pl.BlockSpec((NC, block_rows, d), lambda i: (0, i, 0)),
            pl.BlockSpec((block_rows, d), lambda i: (i, 0)),
            pl.BlockSpec((block_rows, 1), lambda i: (i, 0)),
            pl.BlockSpec((1, d), lambda i: (0, 0)),
        ],
        out_specs=pl.BlockSpec((block_rows, d), lambda i: (i, 0)),
        out_shape=jax.ShapeDtypeStruct((n, d), F32),
    )(acc, y, dinv, b2)


def kernel(ins, edge_index, W, b):
    n, d_in = ins.shape
    d_out = W.shape[1]
    e = edge_index.shape[1]

    chunk = 125
    nchunk = e // (NW * chunk)
    rows_per_tile = n // NS

    src = edge_index[0]
    dst = edge_index[1]
    src3 = src.reshape(NW, nchunk, chunk)
    dst3 = dst.reshape(NW, nchunk, chunk)

    hist = _make_sc_hist(n, e)(dst)
    y, dinv = _tc_prep(hist, ins, W, block_rows=1000)
    zeros = jnp.zeros((rows_per_tile, d_out), F32)
    acc4 = _make_sc_agg(n, e, d_out, chunk, nchunk)(src3, dst3, y, zeros)
    acc = acc4.reshape(NC, n, d_out)
    return _tc_final(acc, y, dinv, b.reshape(1, d_out), block_rows=1000)


# speedup vs baseline: 15.9108x; 15.9108x over previous
import functools

import jax
import jax.numpy as jnp
from jax import lax
from jax.experimental import pallas as pl
from jax.experimental.pallas import tpu as pltpu
from jax.experimental.pallas import tpu_sc as plsc

NC = 2
NS = 16
NW = NC * NS

F32 = jnp.float32
I32 = jnp.int32

CHUNK = 128
HW = 128


def _make_sc_hist(n_pad, nchunk):
    zrows = n_pad // NS
    drows = (n_pad - NS) // NS
    mesh = plsc.VectorSubcoreMesh(core_axis_name="c", subcore_axis_name="s")

    @functools.partial(
        pl.kernel,
        mesh=mesh,
        out_type=jax.ShapeDtypeStruct((NC, NS, drows, HW), F32),
        scratch_types=[
            pltpu.VMEM((nchunk, CHUNK), I32),
            pltpu.VMEM((CHUNK, HW), F32),
            pltpu.VMEM_SHARED((n_pad, HW), F32),
        ],
    )
    def hist_kernel(dst_hbm, ones_hbm, zeros_hbm, out_hbm,
                    dst_v, ones_v, hist_sh):
        cid = lax.axis_index("c")
        sid = lax.axis_index("s")
        wid = sid * NC + cid
        pltpu.sync_copy(dst_hbm.at[wid], dst_v)
        pltpu.sync_copy(ones_hbm, ones_v)
        pltpu.sync_copy(zeros_hbm, hist_sh.at[pl.ds(sid * zrows, zrows)])
        plsc.subcore_barrier()

        def body(j, carry):
            pltpu.sync_copy(ones_v, hist_sh.at[dst_v.at[j]], add=True)
            return carry

        lax.fori_loop(0, nchunk, body, 0)

        plsc.subcore_barrier()
        pltpu.sync_copy(hist_sh.at[pl.ds(sid * drows, drows)],
                        out_hbm.at[cid, sid])

    return hist_kernel


def _tc_prep_body(hist_ref, ins_ref, w_ref, y_ref, dinv_ref):
    h = hist_ref[...]
    deg = h[0, :, :1] + h[1, :, :1] + 1.0
    dinv = lax.rsqrt(deg)
    x = jnp.dot(ins_ref[...], w_ref[...], preferred_element_type=F32)
    y_ref[...] = x * dinv
    dinv_ref[...] = dinv


def _tc_prep(hist2, ins, W, block_rows):
    n, d_in = ins.shape
    d_out = W.shape[1]
    grid = n // block_rows
    return pl.pallas_call(
        _tc_prep_body,
        grid=(grid,),
        in_specs=[
            pl.BlockSpec((NC, block_rows, HW), lambda i: (0, i, 0)),
            pl.BlockSpec((block_rows, d_in), lambda i: (i, 0)),
            pl.BlockSpec((d_in, d_out), lambda i: (0, 0)),
        ],
        out_specs=[
            pl.BlockSpec((block_rows, d_out), lambda i: (i, 0)),
            pl.BlockSpec((block_rows, 1), lambda i: (i, 0)),
        ],
        out_shape=[
            jax.ShapeDtypeStruct((n, d_out), F32),
            jax.ShapeDtypeStruct((n, 1), F32),
        ],
    )(hist2, ins, W)


def _make_sc_agg(n_pad, d, nchunk):
    zrows = n_pad // NS
    drows = (n_pad - NS) // NS
    mesh = plsc.VectorSubcoreMesh(core_axis_name="c", subcore_axis_name="s")

    @functools.partial(
        pl.kernel,
        mesh=mesh,
        out_type=jax.ShapeDtypeStruct((NC, NS, drows, d), F32),
        scratch_types=[
            pltpu.VMEM((nchunk, CHUNK), I32),
            pltpu.VMEM((CHUNK,), I32),
            pltpu.VMEM((CHUNK,), I32),
            pltpu.VMEM((2, CHUNK, d), F32),
            pltpu.VMEM_SHARED((n_pad, d), F32),
            pltpu.SemaphoreType.DMA,
            pltpu.SemaphoreType.DMA,
            pltpu.SemaphoreType.DMA,
            pltpu.SemaphoreType.DMA,
        ],
    )
    def agg_kernel(src_hbm, dst_hbm, y_hbm, zeros_hbm, out_hbm,
                   src_v, dst0_v, dst1_v, rows_v, acc_sh,
                   g0, g1, d0, d1):
        rows0_v = rows_v.at[0]
        rows1_v = rows_v.at[1]
        cid = lax.axis_index("c")
        sid = lax.axis_index("s")
        wid = sid * NC + cid

        pltpu.sync_copy(src_hbm.at[wid], src_v)
        pltpu.sync_copy(zeros_hbm, acc_sh.at[pl.ds(sid * zrows, zrows)])
        plsc.subcore_barrier()
        def body(j, carry):
            pltpu.sync_copy(dst_hbm.at[wid, j], dst0_v)
            pltpu.async_copy(y_hbm.at[src_v.at[j]], rows0_v, g0).wait()
            pltpu.sync_copy(rows0_v, acc_sh.at[dst0_v], add=True)
            return carry

        lax.fori_loop(0, nchunk, body, 0)

        plsc.subcore_barrier()
        pltpu.sync_copy(acc_sh.at[pl.ds(sid * drows, drows)],
                        out_hbm.at[cid, sid])

    return agg_kernel


def _tc_final_body(acc_ref, y_ref, dinv_ref, b_ref, out_ref):
    s = acc_ref[0] + acc_ref[1] + y_ref[...]
    out_ref[...] = jnp.maximum(s * dinv_ref[...] + b_ref[...], 0.0)


def _tc_final(acc, y, dinv, b2, block_rows):
    n, d = y.shape
    grid = n // block_rows
    return pl.pallas_call(
        _tc_final_body,
        grid=(grid,),
        in_specs=[
            pl.BlockSpec((NC, block_rows, d), lambda i: (0, i, 0)),
            pl.BlockSpec((block_rows, d), lambda i: (i, 0)),
            pl.BlockSpec((block_rows, 1), lambda i: (i, 0)),
            pl.BlockSpec((1, d), lambda i: (0, 0)),
        ],
        out_specs=pl.BlockSpec((block_rows, d), lambda i: (i, 0)),
        out_shape=jax.ShapeDtypeStruct((n, d), F32),
    )(acc, y, dinv, b2)


def kernel(ins, edge_index, W, b):
    n, d_in = ins.shape
    d_out = W.shape[1]
    e = edge_index.shape[1]

    e_pad = -(-e // (NW * CHUNK)) * NW * CHUNK
    nchunk = e_pad // (NW * CHUNK)
    n_pad = NS * (-(-(n + 1) // NS))
    drows = (n_pad - NS) // NS
    assert drows * NS == n

    src = edge_index[0]
    dst = edge_index[1]
    pad = e_pad - e
    src3 = jnp.concatenate([src, jnp.zeros((pad,), I32)]).reshape(
        NW, nchunk, CHUNK)
    dst3 = jnp.concatenate([dst, jnp.full((pad,), n, I32)]).reshape(
        NW, nchunk, CHUNK)

    ones16 = jnp.ones((CHUNK, HW), F32)
    zeros16 = jnp.zeros((n_pad // NS, HW), F32)
    hist4 = _make_sc_hist(n_pad, nchunk)(dst3, ones16, zeros16)
    hist2 = hist4.reshape(NC, n, HW)

    y, dinv = _tc_prep(hist2, ins, W, block_rows=1000)

    zeros = jnp.zeros((n_pad // NS, d_out), F32)
    acc4 = _make_sc_agg(n_pad, d_out, nchunk)(src3, dst3, y, zeros)
    acc = acc4.reshape(NC, n, d_out)

    return _tc_final(acc, y, dinv, b.reshape(1, d_out), block_rows=1000)


# speedup vs baseline: 16.9369x; 1.0645x over previous
import functools

import jax
import jax.numpy as jnp
from jax import lax
from jax.experimental import pallas as pl
from jax.experimental.pallas import tpu as pltpu
from jax.experimental.pallas import tpu_sc as plsc

NC = 2
NS = 16
NW = NC * NS

F32 = jnp.float32
I32 = jnp.int32

CHUNK = 128
HW = 128


def _make_sc_hist(n_pad, nchunk):
    zrows = n_pad // NS
    drows = (n_pad - NS) // NS
    mesh = plsc.VectorSubcoreMesh(core_axis_name="c", subcore_axis_name="s")

    @functools.partial(
        pl.kernel,
        mesh=mesh,
        out_type=jax.ShapeDtypeStruct((NC, NS, drows, HW), F32),
        scratch_types=[
            pltpu.VMEM((nchunk, CHUNK), I32),
            pltpu.VMEM((CHUNK, HW), F32),
            pltpu.VMEM_SHARED((n_pad, HW), F32),
        ],
    )
    def hist_kernel(dst_hbm, ones_hbm, zeros_hbm, out_hbm,
                    dst_v, ones_v, hist_sh):
        cid = lax.axis_index("c")
        sid = lax.axis_index("s")
        wid = sid * NC + cid
        pltpu.sync_copy(dst_hbm.at[wid], dst_v)
        pltpu.sync_copy(ones_hbm, ones_v)
        pltpu.sync_copy(zeros_hbm, hist_sh.at[pl.ds(sid * zrows, zrows)])
        plsc.subcore_barrier()

        def body(j, carry):
            pltpu.sync_copy(ones_v, hist_sh.at[dst_v.at[j]], add=True)
            return carry

        lax.fori_loop(0, nchunk, body, 0)

        plsc.subcore_barrier()
        pltpu.sync_copy(hist_sh.at[pl.ds(sid * drows, drows)],
                        out_hbm.at[cid, sid])

    return hist_kernel


def _tc_prep_body(hist_ref, ins_ref, w_ref, y_ref, dinv_ref):
    h = hist_ref[...]
    deg = h[0, :, :1] + h[1, :, :1] + 1.0
    dinv = lax.rsqrt(deg)
    x = jnp.dot(ins_ref[...], w_ref[...], preferred_element_type=F32)
    y_ref[...] = x * dinv
    dinv_ref[...] = dinv


def _tc_prep(hist2, ins, W, block_rows):
    n, d_in = ins.shape
    d_out = W.shape[1]
    grid = n // block_rows
    return pl.pallas_call(
        _tc_prep_body,
        grid=(grid,),
        in_specs=[
            pl.BlockSpec((NC, block_rows, HW), lambda i: (0, i, 0)),
            pl.BlockSpec((block_rows, d_in), lambda i: (i, 0)),
            pl.BlockSpec((d_in, d_out), lambda i: (0, 0)),
        ],
        out_specs=[
            pl.BlockSpec((block_rows, d_out), lambda i: (i, 0)),
            pl.BlockSpec((block_rows, 1), lambda i: (i, 0)),
        ],
        out_shape=[
            jax.ShapeDtypeStruct((n, d_out), F32),
            jax.ShapeDtypeStruct((n, 1), F32),
        ],
    )(hist2, ins, W)


def _make_sc_agg(n_pad, d, nchunk):
    zrows = n_pad // NS
    drows = (n_pad - NS) // NS
    mesh = plsc.VectorSubcoreMesh(core_axis_name="c", subcore_axis_name="s")

    @functools.partial(
        pl.kernel,
        mesh=mesh,
        out_type=jax.ShapeDtypeStruct((NC, NS, drows, d), F32),
        scratch_types=[
            pltpu.VMEM((nchunk, CHUNK), I32),
            pltpu.VMEM((nchunk, CHUNK), I32),
            pltpu.VMEM((CHUNK, d), F32),
            pltpu.VMEM_SHARED((n_pad, d), F32),
            pltpu.SemaphoreType.DMA,
        ],
    )
    def agg_kernel(src_hbm, dst_hbm, y_hbm, zeros_hbm, out_hbm,
                   src_v, dst_v, rows0_v, acc_sh, g0):
        cid = lax.axis_index("c")
        sid = lax.axis_index("s")
        wid = sid * NC + cid

        pltpu.sync_copy(src_hbm.at[wid], src_v)
        pltpu.sync_copy(dst_hbm.at[wid], dst_v)
        pltpu.sync_copy(zeros_hbm, acc_sh.at[pl.ds(sid * zrows, zrows)])
        plsc.subcore_barrier()

        def body(j, carry):
            pltpu.async_copy(y_hbm.at[src_v.at[j]], rows0_v, g0).wait()
            pltpu.sync_copy(rows0_v, acc_sh.at[dst_v.at[j]], add=True)
            return carry

        lax.fori_loop(0, nchunk, body, 0)

        plsc.subcore_barrier()
        pltpu.sync_copy(acc_sh.at[pl.ds(sid * drows, drows)],
                        out_hbm.at[cid, sid])

    return agg_kernel


def _tc_final_body(acc_ref, y_ref, dinv_ref, b_ref, out_ref):
    s = acc_ref[0] + acc_ref[1] + y_ref[...]
    out_ref[...] = jnp.maximum(s * dinv_ref[...] + b_ref[...], 0.0)


def _tc_final(acc, y, dinv, b2, block_rows):
    n, d = y.shape
    grid = n // block_rows
    return pl.pallas_call(
        _tc_final_body,
        grid=(grid,),
        in_specs=[
            pl.BlockSpec((NC, block_rows, d), lambda i: (0, i, 0)),
            pl.BlockSpec((block_rows, d), lambda i: (i, 0)),
            pl.BlockSpec((block_rows, 1), lambda i: (i, 0)),
            pl.BlockSpec((1, d), lambda i: (0, 0)),
        ],
        out_specs=pl.BlockSpec((block_rows, d), lambda i: (i, 0)),
        out_shape=jax.ShapeDtypeStruct((n, d), F32),
    )(acc, y, dinv, b2)


def kernel(ins, edge_index, W, b):
    n, d_in = ins.shape
    d_out = W.shape[1]
    e = edge_index.shape[1]

    e_pad = -(-e // (NW * CHUNK)) * NW * CHUNK
    nchunk = e_pad // (NW * CHUNK)
    n_pad = NS * (-(-(n + 1) // NS))
    drows = (n_pad - NS) // NS
    assert drows * NS == n

    src = edge_index[0]
    dst = edge_index[1]
    pad = e_pad - e
    src3 = jnp.concatenate([src, jnp.zeros((pad,), I32)]).reshape(
        NW, nchunk, CHUNK)
    dst3 = jnp.concatenate([dst, jnp.full((pad,), n, I32)]).reshape(
        NW, nchunk, CHUNK)

    ones16 = jnp.ones((CHUNK, HW), F32)
    zeros16 = jnp.zeros((n_pad // NS, HW), F32)
    hist4 = _make_sc_hist(n_pad, nchunk)(dst3, ones16, zeros16)
    hist2 = hist4.reshape(NC, n, HW)

    y, dinv = _tc_prep(hist2, ins, W, block_rows=1000)

    zeros = jnp.zeros((n_pad // NS, d_out), F32)
    acc4 = _make_sc_agg(n_pad, d_out, nchunk)(src3, dst3, y, zeros)
    acc = acc4.reshape(NC, n, d_out)

    return _tc_final(acc, y, dinv, b.reshape(1, d_out), block_rows=1000)


# speedup vs baseline: 17.5290x; 1.0350x over previous
import functools

import jax
import jax.numpy as jnp
from jax import lax
from jax.experimental import pallas as pl
from jax.experimental.pallas import tpu as pltpu
from jax.experimental.pallas import tpu_sc as plsc

NC = 2
NS = 16
NW = NC * NS

F32 = jnp.float32
I32 = jnp.int32

CHUNK = 128
HW = 32


def _make_sc_hist(n_pad, nchunk):
    zrows = n_pad // NS
    drows = (n_pad - NS) // NS
    mesh = plsc.VectorSubcoreMesh(core_axis_name="c", subcore_axis_name="s")

    @functools.partial(
        pl.kernel,
        mesh=mesh,
        out_type=jax.ShapeDtypeStruct((NC, NS, drows, HW), F32),
        scratch_types=[
            pltpu.VMEM((nchunk, CHUNK), I32),
            pltpu.VMEM((CHUNK, HW), F32),
            pltpu.VMEM_SHARED((n_pad, HW), F32),
        ],
    )
    def hist_kernel(dst_hbm, ones_hbm, zeros_hbm, out_hbm,
                    dst_v, ones_v, hist_sh):
        cid = lax.axis_index("c")
        sid = lax.axis_index("s")
        wid = sid * NC + cid
        pltpu.sync_copy(dst_hbm.at[wid], dst_v)
        pltpu.sync_copy(ones_hbm, ones_v)
        pltpu.sync_copy(zeros_hbm, hist_sh.at[pl.ds(sid * zrows, zrows)])
        plsc.subcore_barrier()

        def body(j, carry):
            pltpu.sync_copy(ones_v, hist_sh.at[dst_v.at[j]], add=True)
            return carry

        lax.fori_loop(0, nchunk, body, 0)

        plsc.subcore_barrier()
        pltpu.sync_copy(hist_sh.at[pl.ds(sid * drows, drows)],
                        out_hbm.at[cid, sid])

    return hist_kernel


def _tc_prep_body(hist_ref, ins_ref, w_ref, y_ref, dinv_ref):
    h = hist_ref[...]
    deg = h[0, :, :1] + h[1, :, :1] + 1.0
    dinv = lax.rsqrt(deg)
    x = jnp.dot(ins_ref[...], w_ref[...], preferred_element_type=F32)
    y_ref[...] = x * dinv
    dinv_ref[...] = dinv


def _tc_prep(hist2, ins, W, block_rows):
    n, d_in = ins.shape
    d_out = W.shape[1]
    grid = n // block_rows
    return pl.pallas_call(
        _tc_prep_body,
        grid=(grid,),
        in_specs=[
            pl.BlockSpec((NC, block_rows, HW), lambda i: (0, i, 0)),
            pl.BlockSpec((block_rows, d_in), lambda i: (i, 0)),
            pl.BlockSpec((d_in, d_out), lambda i: (0, 0)),
        ],
        out_specs=[
            pl.BlockSpec((block_rows, d_out), lambda i: (i, 0)),
            pl.BlockSpec((block_rows, 1), lambda i: (i, 0)),
        ],
        out_shape=[
            jax.ShapeDtypeStruct((n, d_out), F32),
            jax.ShapeDtypeStruct((n, 1), F32),
        ],
    )(hist2, ins, W)


def _make_sc_agg(n_pad, d, nchunk):
    zrows = n_pad // NS
    drows = (n_pad - NS) // NS
    mesh = plsc.VectorSubcoreMesh(core_axis_name="c", subcore_axis_name="s")

    @functools.partial(
        pl.kernel,
        mesh=mesh,
        out_type=jax.ShapeDtypeStruct((NC, NS, drows, d), F32),
        scratch_types=[
            pltpu.VMEM((nchunk, CHUNK), I32),
            pltpu.VMEM((nchunk, CHUNK), I32),
            pltpu.VMEM((CHUNK, d), F32),
            pltpu.VMEM_SHARED((n_pad, d), F32),
            pltpu.SemaphoreType.DMA,
        ],
    )
    def agg_kernel(src_hbm, dst_hbm, y_hbm, zeros_hbm, out_hbm,
                   src_v, dst_v, rows0_v, acc_sh, g0):
        cid = lax.axis_index("c")
        sid = lax.axis_index("s")
        wid = sid * NC + cid

        pltpu.sync_copy(src_hbm.at[wid], src_v)
        pltpu.sync_copy(dst_hbm.at[wid], dst_v)
        pltpu.sync_copy(zeros_hbm, acc_sh.at[pl.ds(sid * zrows, zrows)])
        plsc.subcore_barrier()

        def body(j, carry):
            pltpu.async_copy(y_hbm.at[src_v.at[j]], rows0_v, g0).wait()
            pltpu.sync_copy(rows0_v, acc_sh.at[dst_v.at[j]], add=True)
            return carry

        lax.fori_loop(0, nchunk, body, 0)

        plsc.subcore_barrier()
        pltpu.sync_copy(acc_sh.at[pl.ds(sid * drows, drows)],
                        out_hbm.at[cid, sid])

    return agg_kernel


def _tc_final_body(acc_ref, y_ref, dinv_ref, b_ref, out_ref):
    s = acc_ref[0] + acc_ref[1] + y_ref[...]
    out_ref[...] = jnp.maximum(s * dinv_ref[...] + b_ref[...], 0.0)


def _tc_final(acc, y, dinv, b2, block_rows):
    n, d = y.shape
    grid = n // block_rows
    return pl.pallas_call(
        _tc_final_body,
        grid=(grid,),
        in_specs=[
            pl.BlockSpec((NC, block_rows, d), lambda i: (0, i, 0)),
            pl.BlockSpec((block_rows, d), lambda i: (i, 0)),
            pl.BlockSpec((block_rows, 1), lambda i: (i, 0)),
            pl.BlockSpec((1, d), lambda i: (0, 0)),
        ],
        out_specs=pl.BlockSpec((block_rows, d), lambda i: (i, 0)),
        out_shape=jax.ShapeDtypeStruct((n, d), F32),
    )(acc, y, dinv, b2)


def kernel(ins, edge_index, W, b):
    n, d_in = ins.shape
    d_out = W.shape[1]
    e = edge_index.shape[1]

    e_pad = -(-e // (NW * CHUNK)) * NW * CHUNK
    nchunk = e_pad // (NW * CHUNK)
    n_pad = NS * (-(-(n + 1) // NS))
    drows = (n_pad - NS) // NS
    assert drows * NS == n

    src = edge_index[0]
    dst = edge_index[1]
    pad = e_pad - e
    src3 = jnp.concatenate([src, jnp.zeros((pad,), I32)]).reshape(
        NW, nchunk, CHUNK)
    dst3 = jnp.concatenate([dst, jnp.full((pad,), n, I32)]).reshape(
        NW, nchunk, CHUNK)

    ones16 = jnp.ones((CHUNK, HW), F32)
    zeros16 = jnp.zeros((n_pad // NS, HW), F32)
    hist4 = _make_sc_hist(n_pad, nchunk)(dst3, ones16, zeros16)
    hist2 = hist4.reshape(NC, n, HW)

    y, dinv = _tc_prep(hist2, ins, W, block_rows=1000)

    zeros = jnp.zeros((n_pad // NS, d_out), F32)
    acc4 = _make_sc_agg(n_pad, d_out, nchunk)(src3, dst3, y, zeros)
    acc = acc4.reshape(NC, n, d_out)

    return _tc_final(acc, y, dinv, b.reshape(1, d_out), block_rows=1000)
